# Initial kernel scaffold; baseline (speedup 1.0000x reference)
import jax
import jax.numpy as jnp
from jax.experimental import pallas as pl


def kernel(x, edge_index, batch, W1, b1, W2, b2, LW1, Lb1, LW2, Lb2):
    raise NotImplementedError("write your pallas kernel here")


# speedup vs baseline: 5.4098x; 5.4098x over previous
import functools
import jax
import jax.numpy as jnp
from jax import lax
from jax.experimental import pallas as pl
from jax.experimental.pallas import tpu as pltpu
from jax.experimental.pallas import tpu_sc as plsc

_N = 100000
_E = 1600000
_H = 64
_G = 64
_NC, _NS = 2, 16

_EPAD = 1638400
_NBLK = _EPAD // 128
_BPT = _NBLK // (_NC * _NS)
_NPAD = 100352
_JUNK = _NPAD - 1
_RZ = 392
_W = 8
_NGRP = _H // _W

_mesh = plsc.VectorSubcoreMesh(core_axis_name="c", subcore_axis_name="s")


@functools.partial(
    pl.kernel,
    out_type=jax.ShapeDtypeStruct((_NC, _NPAD, _W), jnp.float32),
    mesh=_mesh,
    compiler_params=pltpu.CompilerParams(use_tc_tiling_on_sc=False),
    scratch_types=[
        pltpu.VMEM((16, 128), jnp.int32),
        pltpu.VMEM((128, _W), jnp.float32),
        pltpu.VMEM((_RZ, _W), jnp.float32),
        pltpu.VMEM_SHARED((_NPAD, _W), jnp.float32),
        pltpu.SemaphoreType.DMA,
    ],
)
def _sc_deg(dst_h, oh_h, z_h, out_h, idxv, ohv, zv, acc, sem):
    core = lax.axis_index("c")
    s = lax.axis_index("s")
    pltpu.sync_copy(z_h, zv)
    pltpu.sync_copy(oh_h, ohv)
    for kk in range(16):
        pltpu.sync_copy(zv, acc.at[pl.ds((s * 16 + kk) * _RZ, _RZ), :])
    plsc.subcore_barrier()
    blk0 = (core * _NS + s) * _BPT

    def body(i, carry):
        pltpu.sync_copy(dst_h.at[pl.ds(blk0 + i * 16, 16), :], idxv)
        for j in range(16):
            pltpu.sync_copy(ohv, acc.at[idxv.at[j]], add=True)
        return carry

    lax.fori_loop(0, _BPT // 16, body, 0)
    plsc.subcore_barrier()
    rpt = _NPAD // _NS
    pltpu.sync_copy(acc.at[pl.ds(s * rpt, rpt), :],
                    out_h.at[core, pl.ds(s * rpt, rpt), :])


@functools.partial(
    pl.kernel,
    out_type=[jax.ShapeDtypeStruct((_NC, _NPAD, _W), jnp.float32)] * _NGRP,
    mesh=_mesh,
    compiler_params=pltpu.CompilerParams(use_tc_tiling_on_sc=False),
    scratch_types=[
        pltpu.VMEM((16, 128), jnp.int32),
        pltpu.VMEM((16, 128), jnp.int32),
        pltpu.VMEM((8, 128, _W), jnp.float32),
        pltpu.VMEM((_RZ, _W), jnp.float32),
        pltpu.VMEM_SHARED((_NPAD, _W), jnp.float32),
        pltpu.SemaphoreType.DMA,
    ],
)
def _sc_agg(src_h, dst_h, t0h, t1h, t2h, t3h, t4h, t5h, t6h, t7h, z_h,
            o0, o1, o2, o3, o4, o5, o6, o7,
            srcv, dstv, rows, zv, acc, gsem):
    core = lax.axis_index("c")
    s = lax.axis_index("s")
    pltpu.sync_copy(z_h, zv)
    hbpt = _NBLK // 2 // _NS
    blk0 = core * (_NBLK // 2) + s * hbpt
    rpt = _NPAD // _NS

    for th, oh in ((t0h, o0), (t1h, o1), (t2h, o2), (t3h, o3),
                   (t4h, o4), (t5h, o5), (t6h, o6), (t7h, o7)):
        for kk in range(16):
            pltpu.sync_copy(zv, acc.at[pl.ds((s * 16 + kk) * _RZ, _RZ), :])
        plsc.subcore_barrier()

        def body(i, carry):
            pltpu.sync_copy(src_h.at[pl.ds(blk0 + i * 16, 16), :], srcv)
            pltpu.sync_copy(dst_h.at[pl.ds(blk0 + i * 16, 16), :], dstv)
            for h in range(2):
                cps = [pltpu.async_copy(th.at[srcv.at[h * 8 + j]],
                                        rows.at[j], gsem)
                       for j in range(8)]
                for cp in cps:
                    cp.wait()
                for j in range(8):
                    pltpu.sync_copy(rows.at[j],
                                    acc.at[dstv.at[h * 8 + j]], add=True)
            return carry

        lax.fori_loop(0, hbpt // 16, body, 0)
        plsc.subcore_barrier()
        pltpu.sync_copy(acc.at[pl.ds(s * rpt, rpt), :],
                        oh.at[core, pl.ds(s * rpt, rpt), :])
        plsc.subcore_barrier()


_B = 1000
_NB = _N // _B


def _tc1_body(x_ref, da_ref, db_ref, w1_ref, *outs):
    os_, dv_ref = outs[:_NGRP], outs[_NGRP]
    deg = da_ref[...] + db_ref[...] + 1.0
    dinv = lax.rsqrt(deg)
    h = jnp.dot(x_ref[...], w1_ref[...], preferred_element_type=jnp.float32)
    hp = h * dinv
    dv_ref[...] = dinv
    for g in range(_NGRP):
        os_[g][...] = hp[:, g * _W:(g + 1) * _W]


def _tc2_body(*refs):
    ags = refs[:_NGRP]
    hgs = refs[_NGRP:2 * _NGRP]
    dv_ref, b1_ref, w2_ref = refs[2 * _NGRP:2 * _NGRP + 3]
    os_ = refs[2 * _NGRP + 3:]
    dinv = dv_ref[...]
    parts = []
    for ag, hg in zip(ags, hgs):
        a = ag[...]
        parts.append(a[0] + a[1] + hg[...])
    z = jnp.concatenate(parts, axis=1) * dinv + b1_ref[...]
    z = jnp.maximum(z, 0.0)
    h2o = jnp.dot(z, w2_ref[...], preferred_element_type=jnp.float32)
    hp = h2o * dinv
    for g in range(_NGRP):
        os_[g][...] = hp[:, g * _W:(g + 1) * _W]


def _tc3_body(*refs):
    ags = refs[:_NGRP]
    hgs = refs[_NGRP:2 * _NGRP]
    (dv_ref, b2_ref, bat_ref, lw1_ref, lb1_ref, lw2_ref, lb2_ref,
     out_ref, sums_ref, cnt_ref) = refs[2 * _NGRP:]
    i = pl.program_id(0)
    dinv = dv_ref[...]
    parts = []
    for ag, hg in zip(ags, hgs):
        a = ag[...]
        parts.append(a[0] + a[1] + hg[...])
    z = jnp.concatenate(parts, axis=1) * dinv + b2_ref[...]
    z = jnp.maximum(z, 0.0)
    gid = lax.broadcasted_iota(jnp.int32, (_B, _G), 1)
    oh = (bat_ref[...] == gid).astype(jnp.float32)
    dn = (((0,), (0,)), ((), ()))
    psum = lax.dot_general(oh, z, dn, preferred_element_type=jnp.float32)
    pcnt = lax.dot_general(oh, jnp.ones((_B, 1), jnp.float32), dn,
                           preferred_element_type=jnp.float32)

    @pl.when(i == 0)
    def _():
        sums_ref[...] = psum
        cnt_ref[...] = pcnt

    @pl.when(i > 0)
    def _():
        sums_ref[...] += psum
        cnt_ref[...] += pcnt

    @pl.when(i == _NB - 1)
    def _():
        p = sums_ref[...] / jnp.maximum(cnt_ref[...], 1.0)
        q = jnp.dot(p, lw1_ref[...], preferred_element_type=jnp.float32)
        q = jnp.maximum(q + lb1_ref[...], 0.0)
        out_ref[...] = jnp.dot(q, lw2_ref[...],
                               preferred_element_type=jnp.float32) + lb2_ref[...]


def _row_spec(w):
    return pl.BlockSpec((_B, w), lambda i: (i, 0))


def _agg_spec():
    return pl.BlockSpec((_NC, _B, _W), lambda i: (0, i, 0))


def _full_spec(shape):
    nd = len(shape)
    return pl.BlockSpec(shape, lambda i: (0,) * nd)


def kernel(x, edge_index, batch, W1, b1, W2, b2, LW1, Lb1, LW2, Lb2):
    src = edge_index[0]
    dst = edge_index[1]
    pad = _EPAD - _E
    src2 = jnp.concatenate([src, jnp.zeros((pad,), jnp.int32)]).reshape(_NBLK, 128)
    dst2 = jnp.concatenate([dst, jnp.full((pad,), _JUNK, jnp.int32)]).reshape(_NBLK, 128)

    zeros_w = jnp.zeros((_RZ, _W), jnp.float32)
    onehot_rows = jnp.zeros((128, _W), jnp.float32).at[:, 0].set(1.0)

    deg_out = _sc_deg(dst2, onehot_rows, zeros_w)
    degA = deg_out[0, :_N, 0:1]
    degB = deg_out[1, :_N, 0:1]

    t1 = pl.pallas_call(
        _tc1_body,
        grid=(_NB,),
        in_specs=[_row_spec(11), _row_spec(1), _row_spec(1), _full_spec((11, _H))],
        out_specs=[_row_spec(_W)] * _NGRP + [_row_spec(1)],
        out_shape=[jax.ShapeDtypeStruct((_N, _W), jnp.float32)] * _NGRP
        + [jax.ShapeDtypeStruct((_N, 1), jnp.float32)],
    )(x, degA, degB, W1)
    h1g, dinv = t1[:_NGRP], t1[_NGRP]

    ag1 = _sc_agg(src2, dst2, *h1g, zeros_w)

    h2g = pl.pallas_call(
        _tc2_body,
        grid=(_NB,),
        in_specs=[_agg_spec()] * _NGRP + [_row_spec(_W)] * _NGRP
        + [_row_spec(1), _full_spec((1, _H)), _full_spec((_H, _H))],
        out_specs=[_row_spec(_W)] * _NGRP,
        out_shape=[jax.ShapeDtypeStruct((_N, _W), jnp.float32)] * _NGRP,
    )(*ag1, *h1g, dinv, b1.reshape(1, _H), W2)

    ag2 = _sc_agg(src2, dst2, *h2g, zeros_w)

    out = pl.pallas_call(
        _tc3_body,
        grid=(_NB,),
        in_specs=[_agg_spec()] * _NGRP + [_row_spec(_W)] * _NGRP
        + [_row_spec(1), _full_spec((1, _H)), _row_spec(1),
           _full_spec((_H, _H)), _full_spec((1, _H)),
           _full_spec((_H, 1)), _full_spec((1, 1))],
        out_specs=pl.BlockSpec((_G, 1), lambda i: (0, 0)),
        out_shape=jax.ShapeDtypeStruct((_G, 1), jnp.float32),
        scratch_shapes=[pltpu.VMEM((_G, _G), jnp.float32),
                        pltpu.VMEM((_G, 1), jnp.float32)],
    )(*ag2, *h2g, dinv, b2.reshape(1, _H), batch.reshape(_N, 1),
      LW1, Lb1.reshape(1, _H), LW2, Lb2.reshape(1, 1))

    return out


# speedup vs baseline: 5.5700x; 1.0296x over previous
import functools
import jax
import jax.numpy as jnp
from jax import lax
from jax.experimental import pallas as pl
from jax.experimental.pallas import tpu as pltpu
from jax.experimental.pallas import tpu_sc as plsc

_N = 100000
_E = 1600000
_H = 64
_G = 64
_NC, _NS = 2, 16

_EPAD = 1638400
_NBLK = _EPAD // 128
_BPT = _NBLK // (_NC * _NS)
_NPAD = 100352
_JUNK = _NPAD - 1
_RZ = 392
_W = 8
_NGRP = _H // _W

_mesh = plsc.VectorSubcoreMesh(core_axis_name="c", subcore_axis_name="s")


@functools.partial(
    pl.kernel,
    out_type=jax.ShapeDtypeStruct((_NC, _NPAD, _W), jnp.float32),
    mesh=_mesh,
    compiler_params=pltpu.CompilerParams(use_tc_tiling_on_sc=False),
    scratch_types=[
        pltpu.VMEM((16, 128), jnp.int32),
        pltpu.VMEM((128, _W), jnp.float32),
        pltpu.VMEM((_RZ, _W), jnp.float32),
        pltpu.VMEM_SHARED((_NPAD, _W), jnp.float32),
        pltpu.SemaphoreType.DMA,
    ],
)
def _sc_deg(dst_h, oh_h, z_h, out_h, idxv, ohv, zv, acc, sem):
    core = lax.axis_index("c")
    s = lax.axis_index("s")
    pltpu.sync_copy(z_h, zv)
    pltpu.sync_copy(oh_h, ohv)
    for kk in range(16):
        pltpu.sync_copy(zv, acc.at[pl.ds((s * 16 + kk) * _RZ, _RZ), :])
    plsc.subcore_barrier()
    blk0 = (core * _NS + s) * _BPT

    def body(i, carry):
        pltpu.sync_copy(dst_h.at[pl.ds(blk0 + i * 16, 16), :], idxv)
        for j in range(16):
            pltpu.sync_copy(ohv, acc.at[idxv.at[j]], add=True)
        return carry

    lax.fori_loop(0, _BPT // 16, body, 0)
    plsc.subcore_barrier()
    rpt = _NPAD // _NS
    pltpu.sync_copy(acc.at[pl.ds(s * rpt, rpt), :],
                    out_h.at[core, pl.ds(s * rpt, rpt), :])


@functools.partial(
    pl.kernel,
    out_type=[jax.ShapeDtypeStruct((_NC, _NPAD, _W), jnp.float32)] * _NGRP,
    mesh=_mesh,
    compiler_params=pltpu.CompilerParams(use_tc_tiling_on_sc=False),
    scratch_types=[
        pltpu.VMEM((2048,), jnp.int32),
        pltpu.VMEM((2048,), jnp.int32),
        pltpu.VMEM((2048, _W), jnp.float32),
        pltpu.VMEM((_RZ, _W), jnp.float32),
        pltpu.VMEM_SHARED((_NPAD, _W), jnp.float32),
        pltpu.SemaphoreType.DMA,
    ],
)
def _sc_agg(src_h, dst_h, t0h, t1h, t2h, t3h, t4h, t5h, t6h, t7h, z_h,
            o0, o1, o2, o3, o4, o5, o6, o7,
            srcv, dstv, rows, zv, acc, gsem):
    core = lax.axis_index("c")
    s = lax.axis_index("s")
    pltpu.sync_copy(z_h, zv)
    hbpt = _NBLK // 2 // _NS
    blk0 = core * (_NBLK // 2) + s * hbpt
    rpt = _NPAD // _NS

    for th, oh in ((t0h, o0), (t1h, o1), (t2h, o2), (t3h, o3),
                   (t4h, o4), (t5h, o5), (t6h, o6), (t7h, o7)):
        for kk in range(16):
            pltpu.sync_copy(zv, acc.at[pl.ds((s * 16 + kk) * _RZ, _RZ), :])
        plsc.subcore_barrier()

        def body(i, carry):
            eoff = blk0 * 128 + i * 2048
            pltpu.sync_copy(src_h.at[pl.ds(eoff, 2048)], srcv)
            pltpu.sync_copy(dst_h.at[pl.ds(eoff, 2048)], dstv)
            pltpu.async_copy(th.at[srcv], rows, gsem).wait()
            pltpu.sync_copy(rows, acc.at[dstv], add=True)
            return carry

        lax.fori_loop(0, hbpt // 16, body, 0)
        plsc.subcore_barrier()
        pltpu.sync_copy(acc.at[pl.ds(s * rpt, rpt), :],
                        oh.at[core, pl.ds(s * rpt, rpt), :])
        plsc.subcore_barrier()


_B = 1000
_NB = _N // _B


def _tc1_body(x_ref, da_ref, db_ref, w1_ref, *outs):
    os_, dv_ref = outs[:_NGRP], outs[_NGRP]
    deg = da_ref[...] + db_ref[...] + 1.0
    dinv = lax.rsqrt(deg)
    h = jnp.dot(x_ref[...], w1_ref[...], preferred_element_type=jnp.float32)
    hp = h * dinv
    dv_ref[...] = dinv
    for g in range(_NGRP):
        os_[g][...] = hp[:, g * _W:(g + 1) * _W]


def _tc2_body(*refs):
    ags = refs[:_NGRP]
    hgs = refs[_NGRP:2 * _NGRP]
    dv_ref, b1_ref, w2_ref = refs[2 * _NGRP:2 * _NGRP + 3]
    os_ = refs[2 * _NGRP + 3:]
    dinv = dv_ref[...]
    parts = []
    for ag, hg in zip(ags, hgs):
        a = ag[...]
        parts.append(a[0] + a[1] + hg[...])
    z = jnp.concatenate(parts, axis=1) * dinv + b1_ref[...]
    z = jnp.maximum(z, 0.0)
    h2o = jnp.dot(z, w2_ref[...], preferred_element_type=jnp.float32)
    hp = h2o * dinv
    for g in range(_NGRP):
        os_[g][...] = hp[:, g * _W:(g + 1) * _W]


def _tc3_body(*refs):
    ags = refs[:_NGRP]
    hgs = refs[_NGRP:2 * _NGRP]
    (dv_ref, b2_ref, bat_ref, lw1_ref, lb1_ref, lw2_ref, lb2_ref,
     out_ref, sums_ref, cnt_ref) = refs[2 * _NGRP:]
    i = pl.program_id(0)
    dinv = dv_ref[...]
    parts = []
    for ag, hg in zip(ags, hgs):
        a = ag[...]
        parts.append(a[0] + a[1] + hg[...])
    z = jnp.concatenate(parts, axis=1) * dinv + b2_ref[...]
    z = jnp.maximum(z, 0.0)
    gid = lax.broadcasted_iota(jnp.int32, (_B, _G), 1)
    oh = (bat_ref[...] == gid).astype(jnp.float32)
    dn = (((0,), (0,)), ((), ()))
    psum = lax.dot_general(oh, z, dn, preferred_element_type=jnp.float32)
    pcnt = lax.dot_general(oh, jnp.ones((_B, 1), jnp.float32), dn,
                           preferred_element_type=jnp.float32)

    @pl.when(i == 0)
    def _():
        sums_ref[...] = psum
        cnt_ref[...] = pcnt

    @pl.when(i > 0)
    def _():
        sums_ref[...] += psum
        cnt_ref[...] += pcnt

    @pl.when(i == _NB - 1)
    def _():
        p = sums_ref[...] / jnp.maximum(cnt_ref[...], 1.0)
        q = jnp.dot(p, lw1_ref[...], preferred_element_type=jnp.float32)
        q = jnp.maximum(q + lb1_ref[...], 0.0)
        out_ref[...] = jnp.dot(q, lw2_ref[...],
                               preferred_element_type=jnp.float32) + lb2_ref[...]


def _row_spec(w):
    return pl.BlockSpec((_B, w), lambda i: (i, 0))


def _agg_spec():
    return pl.BlockSpec((_NC, _B, _W), lambda i: (0, i, 0))


def _full_spec(shape):
    nd = len(shape)
    return pl.BlockSpec(shape, lambda i: (0,) * nd)


def kernel(x, edge_index, batch, W1, b1, W2, b2, LW1, Lb1, LW2, Lb2):
    src = edge_index[0]
    dst = edge_index[1]
    pad = _EPAD - _E
    srcf = jnp.concatenate([src, jnp.zeros((pad,), jnp.int32)])
    dstf = jnp.concatenate([dst, jnp.full((pad,), _JUNK, jnp.int32)])
    dst2 = dstf.reshape(_NBLK, 128)

    zeros_w = jnp.zeros((_RZ, _W), jnp.float32)
    onehot_rows = jnp.zeros((128, _W), jnp.float32).at[:, 0].set(1.0)

    deg_out = _sc_deg(dst2, onehot_rows, zeros_w)
    degA = deg_out[0, :_N, 0:1]
    degB = deg_out[1, :_N, 0:1]

    t1 = pl.pallas_call(
        _tc1_body,
        grid=(_NB,),
        in_specs=[_row_spec(11), _row_spec(1), _row_spec(1), _full_spec((11, _H))],
        out_specs=[_row_spec(_W)] * _NGRP + [_row_spec(1)],
        out_shape=[jax.ShapeDtypeStruct((_N, _W), jnp.float32)] * _NGRP
        + [jax.ShapeDtypeStruct((_N, 1), jnp.float32)],
    )(x, degA, degB, W1)
    h1g, dinv = t1[:_NGRP], t1[_NGRP]

    ag1 = _sc_agg(srcf, dstf, *h1g, zeros_w)

    h2g = pl.pallas_call(
        _tc2_body,
        grid=(_NB,),
        in_specs=[_agg_spec()] * _NGRP + [_row_spec(_W)] * _NGRP
        + [_row_spec(1), _full_spec((1, _H)), _full_spec((_H, _H))],
        out_specs=[_row_spec(_W)] * _NGRP,
        out_shape=[jax.ShapeDtypeStruct((_N, _W), jnp.float32)] * _NGRP,
    )(*ag1, *h1g, dinv, b1.reshape(1, _H), W2)

    ag2 = _sc_agg(srcf, dstf, *h2g, zeros_w)

    out = pl.pallas_call(
        _tc3_body,
        grid=(_NB,),
        in_specs=[_agg_spec()] * _NGRP + [_row_spec(_W)] * _NGRP
        + [_row_spec(1), _full_spec((1, _H)), _row_spec(1),
           _full_spec((_H, _H)), _full_spec((1, _H)),
           _full_spec((_H, 1)), _full_spec((1, 1))],
        out_specs=pl.BlockSpec((_G, 1), lambda i: (0, 0)),
        out_shape=jax.ShapeDtypeStruct((_G, 1), jnp.float32),
        scratch_shapes=[pltpu.VMEM((_G, _G), jnp.float32),
                        pltpu.VMEM((_G, 1), jnp.float32)],
    )(*ag2, *h2g, dinv, b2.reshape(1, _H), batch.reshape(_N, 1),
      LW1, Lb1.reshape(1, _H), LW2, Lb2.reshape(1, 1))

    return out


# speedup vs baseline: 6.3236x; 1.1353x over previous
import functools
import jax
import jax.numpy as jnp
from jax import lax
from jax.experimental import pallas as pl
from jax.experimental.pallas import tpu as pltpu
from jax.experimental.pallas import tpu_sc as plsc

_N = 100000
_E = 1600000
_H = 64
_G = 64
_NC, _NS = 2, 16

_EPAD = 1638400
_NBLK = _EPAD // 128
_BPT = _NBLK // (_NC * _NS)
_NPAD = 100352
_JUNK = _NPAD - 1
_RZ = 392
_W = 8
_NGRP = _H // _W
_CHK = 1600
_NIT = 32

_mesh = plsc.VectorSubcoreMesh(core_axis_name="c", subcore_axis_name="s")


@functools.partial(
    pl.kernel,
    out_type=jax.ShapeDtypeStruct((_NC, _NPAD, _W), jnp.float32),
    mesh=_mesh,
    compiler_params=pltpu.CompilerParams(use_tc_tiling_on_sc=False),
    scratch_types=[
        pltpu.VMEM((16, 128), jnp.int32),
        pltpu.VMEM((128, _W), jnp.float32),
        pltpu.VMEM((_RZ, _W), jnp.float32),
        pltpu.VMEM_SHARED((_NPAD, _W), jnp.float32),
        pltpu.SemaphoreType.DMA,
    ],
)
def _sc_deg(dst_h, oh_h, z_h, out_h, idxv, ohv, zv, acc, sem):
    core = lax.axis_index("c")
    s = lax.axis_index("s")
    pltpu.sync_copy(z_h, zv)
    pltpu.sync_copy(oh_h, ohv)
    for kk in range(16):
        pltpu.sync_copy(zv, acc.at[pl.ds((s * 16 + kk) * _RZ, _RZ), :])
    plsc.subcore_barrier()
    blk0 = (core * _NS + s) * _BPT

    def body(i, carry):
        pltpu.sync_copy(dst_h.at[pl.ds(blk0 + i * 16, 16), :], idxv)
        for j in range(16):
            pltpu.sync_copy(ohv, acc.at[idxv.at[j]], add=True)
        return carry

    lax.fori_loop(0, _BPT // 16, body, 0)
    plsc.subcore_barrier()
    rpt = _NPAD // _NS
    pltpu.sync_copy(acc.at[pl.ds(s * rpt, rpt), :],
                    out_h.at[core, pl.ds(s * rpt, rpt), :])


@functools.partial(
    pl.kernel,
    out_type=[jax.ShapeDtypeStruct((_NC, _NPAD, _W), jnp.float32)] * _NGRP,
    mesh=_mesh,
    compiler_params=pltpu.CompilerParams(use_tc_tiling_on_sc=False),
    scratch_types=[
        pltpu.VMEM((2, _CHK), jnp.int32),
        pltpu.VMEM((2, _CHK), jnp.int32),
        pltpu.VMEM((2, _CHK, _W), jnp.float32),
        pltpu.VMEM((_RZ, _W), jnp.float32),
        pltpu.VMEM_SHARED((_NPAD, _W), jnp.float32),
        pltpu.SemaphoreType.DMA,
        pltpu.SemaphoreType.DMA,
    ],
)
def _sc_agg(src_h, dst_h, t0h, t1h, t2h, t3h, t4h, t5h, t6h, t7h, z_h,
            o0, o1, o2, o3, o4, o5, o6, o7,
            srcv, dstv, rows, zv, acc, gsem, ssem):
    core = lax.axis_index("c")
    s = lax.axis_index("s")
    pltpu.sync_copy(z_h, zv)
    hbpt = _NBLK // 2 // _NS
    blk0 = core * (_NBLK // 2) + s * hbpt
    rpt = _NPAD // _NS

    for th, oh in ((t0h, o0), (t1h, o1), (t2h, o2), (t3h, o3),
                   (t4h, o4), (t5h, o5), (t6h, o6), (t7h, o7)):
        for kk in range(16):
            pltpu.sync_copy(zv, acc.at[pl.ds((s * 16 + kk) * _RZ, _RZ), :])
        plsc.subcore_barrier()

        e0 = blk0 * 128
        for b in range(2):
            pltpu.sync_copy(src_h.at[pl.ds(e0 + b * _CHK, _CHK)], srcv.at[b])
            pltpu.sync_copy(dst_h.at[pl.ds(e0 + b * _CHK, _CHK)], dstv.at[b])
            pltpu.async_copy(th.at[srcv.at[b]], rows.at[b], gsem)

        def outer(o, carry):
            for b in range(2):
                i = 2 * o + b
                pltpu.make_async_copy(th.at[srcv.at[b]], rows.at[b],
                                      gsem).wait()
                pltpu.async_copy(rows.at[b], acc.at[dstv.at[b]], ssem,
                                 add=True)
                pltpu.make_async_copy(rows.at[b], acc.at[dstv.at[b]],
                                      ssem).wait()

                @pl.when(i + 2 < _NIT)
                def _():
                    eo = e0 + (i + 2) * _CHK
                    pltpu.sync_copy(src_h.at[pl.ds(eo, _CHK)], srcv.at[b])
                    pltpu.sync_copy(dst_h.at[pl.ds(eo, _CHK)], dstv.at[b])
                    pltpu.async_copy(th.at[srcv.at[b]], rows.at[b], gsem)
            return carry

        lax.fori_loop(0, _NIT // 2, outer, 0)
        plsc.subcore_barrier()
        pltpu.sync_copy(acc.at[pl.ds(s * rpt, rpt), :],
                        oh.at[core, pl.ds(s * rpt, rpt), :])
        plsc.subcore_barrier()


_B = 1000
_NB = _N // _B


def _tc1_body(x_ref, da_ref, db_ref, w1_ref, *outs):
    os_, dv_ref = outs[:_NGRP], outs[_NGRP]
    deg = da_ref[...] + db_ref[...] + 1.0
    dinv = lax.rsqrt(deg)
    h = jnp.dot(x_ref[...], w1_ref[...], preferred_element_type=jnp.float32)
    hp = h * dinv
    dv_ref[...] = dinv
    for g in range(_NGRP):
        os_[g][...] = hp[:, g * _W:(g + 1) * _W]


def _tc2_body(*refs):
    ags = refs[:_NGRP]
    hgs = refs[_NGRP:2 * _NGRP]
    dv_ref, b1_ref, w2_ref = refs[2 * _NGRP:2 * _NGRP + 3]
    os_ = refs[2 * _NGRP + 3:]
    dinv = dv_ref[...]
    parts = []
    for ag, hg in zip(ags, hgs):
        a = ag[...]
        parts.append(a[0] + a[1] + hg[...])
    z = jnp.concatenate(parts, axis=1) * dinv + b1_ref[...]
    z = jnp.maximum(z, 0.0)
    h2o = jnp.dot(z, w2_ref[...], preferred_element_type=jnp.float32)
    hp = h2o * dinv
    for g in range(_NGRP):
        os_[g][...] = hp[:, g * _W:(g + 1) * _W]


def _tc3_body(*refs):
    ags = refs[:_NGRP]
    hgs = refs[_NGRP:2 * _NGRP]
    (dv_ref, b2_ref, bat_ref, lw1_ref, lb1_ref, lw2_ref, lb2_ref,
     out_ref, sums_ref, cnt_ref) = refs[2 * _NGRP:]
    i = pl.program_id(0)
    dinv = dv_ref[...]
    parts = []
    for ag, hg in zip(ags, hgs):
        a = ag[...]
        parts.append(a[0] + a[1] + hg[...])
    z = jnp.concatenate(parts, axis=1) * dinv + b2_ref[...]
    z = jnp.maximum(z, 0.0)
    gid = lax.broadcasted_iota(jnp.int32, (_B, _G), 1)
    oh = (bat_ref[...] == gid).astype(jnp.float32)
    dn = (((0,), (0,)), ((), ()))
    psum = lax.dot_general(oh, z, dn, preferred_element_type=jnp.float32)
    pcnt = lax.dot_general(oh, jnp.ones((_B, 1), jnp.float32), dn,
                           preferred_element_type=jnp.float32)

    @pl.when(i == 0)
    def _():
        sums_ref[...] = psum
        cnt_ref[...] = pcnt

    @pl.when(i > 0)
    def _():
        sums_ref[...] += psum
        cnt_ref[...] += pcnt

    @pl.when(i == _NB - 1)
    def _():
        p = sums_ref[...] / jnp.maximum(cnt_ref[...], 1.0)
        q = jnp.dot(p, lw1_ref[...], preferred_element_type=jnp.float32)
        q = jnp.maximum(q + lb1_ref[...], 0.0)
        out_ref[...] = jnp.dot(q, lw2_ref[...],
                               preferred_element_type=jnp.float32) + lb2_ref[...]


def _row_spec(w):
    return pl.BlockSpec((_B, w), lambda i: (i, 0))


def _agg_spec():
    return pl.BlockSpec((_NC, _B, _W), lambda i: (0, i, 0))


def _full_spec(shape):
    nd = len(shape)
    return pl.BlockSpec(shape, lambda i: (0,) * nd)


def kernel(x, edge_index, batch, W1, b1, W2, b2, LW1, Lb1, LW2, Lb2):
    src = edge_index[0]
    dst = edge_index[1]
    pad = _EPAD - _E
    srcf = jnp.concatenate([src, jnp.zeros((pad,), jnp.int32)])
    dstf = jnp.concatenate([dst, jnp.full((pad,), _JUNK, jnp.int32)])
    dst2 = dstf.reshape(_NBLK, 128)

    zeros_w = jnp.zeros((_RZ, _W), jnp.float32)
    onehot_rows = jnp.zeros((128, _W), jnp.float32).at[:, 0].set(1.0)

    deg_out = _sc_deg(dst2, onehot_rows, zeros_w)
    degA = deg_out[0, :_N, 0:1]
    degB = deg_out[1, :_N, 0:1]

    t1 = pl.pallas_call(
        _tc1_body,
        grid=(_NB,),
        in_specs=[_row_spec(11), _row_spec(1), _row_spec(1), _full_spec((11, _H))],
        out_specs=[_row_spec(_W)] * _NGRP + [_row_spec(1)],
        out_shape=[jax.ShapeDtypeStruct((_N, _W), jnp.float32)] * _NGRP
        + [jax.ShapeDtypeStruct((_N, 1), jnp.float32)],
    )(x, degA, degB, W1)
    h1g, dinv = t1[:_NGRP], t1[_NGRP]

    ag1 = _sc_agg(srcf, dstf, *h1g, zeros_w)

    h2g = pl.pallas_call(
        _tc2_body,
        grid=(_NB,),
        in_specs=[_agg_spec()] * _NGRP + [_row_spec(_W)] * _NGRP
        + [_row_spec(1), _full_spec((1, _H)), _full_spec((_H, _H))],
        out_specs=[_row_spec(_W)] * _NGRP,
        out_shape=[jax.ShapeDtypeStruct((_N, _W), jnp.float32)] * _NGRP,
    )(*ag1, *h1g, dinv, b1.reshape(1, _H), W2)

    ag2 = _sc_agg(srcf, dstf, *h2g, zeros_w)

    out = pl.pallas_call(
        _tc3_body,
        grid=(_NB,),
        in_specs=[_agg_spec()] * _NGRP + [_row_spec(_W)] * _NGRP
        + [_row_spec(1), _full_spec((1, _H)), _row_spec(1),
           _full_spec((_H, _H)), _full_spec((1, _H)),
           _full_spec((_H, 1)), _full_spec((1, 1))],
        out_specs=pl.BlockSpec((_G, 1), lambda i: (0, 0)),
        out_shape=jax.ShapeDtypeStruct((_G, 1), jnp.float32),
        scratch_shapes=[pltpu.VMEM((_G, _G), jnp.float32),
                        pltpu.VMEM((_G, 1), jnp.float32)],
    )(*ag2, *h2g, dinv, b2.reshape(1, _H), batch.reshape(_N, 1),
      LW1, Lb1.reshape(1, _H), LW2, Lb2.reshape(1, 1))

    return out


# speedup vs baseline: 11.8972x; 1.8814x over previous
import functools
import jax
import jax.numpy as jnp
from jax import lax
from jax.experimental import pallas as pl
from jax.experimental.pallas import tpu as pltpu
from jax.experimental.pallas import tpu_sc as plsc

_N = 100000
_E = 1600000
_H = 64
_G = 64
_NC, _NS = 2, 16

_EPAD = 1638400
_NBLK = _EPAD // 128
_BPT = _NBLK // (_NC * _NS)
_NPAD = 100352
_JUNK = _NPAD - 1
_RZ = 392
_W = 16
_NGRP = _H // _W
_DW = 8
_CHK = 1600
_NIT = 32

_mesh = plsc.VectorSubcoreMesh(core_axis_name="c", subcore_axis_name="s")


@functools.partial(
    pl.kernel,
    out_type=jax.ShapeDtypeStruct((_NC, _NPAD, _DW), jnp.float32),
    mesh=_mesh,
    compiler_params=pltpu.CompilerParams(use_tc_tiling_on_sc=False),
    scratch_types=[
        pltpu.VMEM((16, 128), jnp.int32),
        pltpu.VMEM((128, _DW), jnp.float32),
        pltpu.VMEM((_RZ, _DW), jnp.float32),
        pltpu.VMEM_SHARED((_NPAD, _DW), jnp.float32),
        pltpu.SemaphoreType.DMA,
    ],
)
def _sc_deg(dst_h, oh_h, z_h, out_h, idxv, ohv, zv, acc, sem):
    core = lax.axis_index("c")
    s = lax.axis_index("s")
    pltpu.sync_copy(z_h, zv)
    pltpu.sync_copy(oh_h, ohv)
    for kk in range(16):
        pltpu.sync_copy(zv, acc.at[pl.ds((s * 16 + kk) * _RZ, _RZ), :])
    plsc.subcore_barrier()
    blk0 = (core * _NS + s) * _BPT

    def body(i, carry):
        pltpu.sync_copy(dst_h.at[pl.ds(blk0 + i * 16, 16), :], idxv)
        for j in range(16):
            pltpu.sync_copy(ohv, acc.at[idxv.at[j]], add=True)
        return carry

    lax.fori_loop(0, _BPT // 16, body, 0)
    plsc.subcore_barrier()
    rpt = _NPAD // _NS
    pltpu.sync_copy(acc.at[pl.ds(s * rpt, rpt), :],
                    out_h.at[core, pl.ds(s * rpt, rpt), :])


@functools.partial(
    pl.kernel,
    out_type=[jax.ShapeDtypeStruct((_NC, _NPAD, _W), jnp.bfloat16)] * _NGRP,
    mesh=_mesh,
    compiler_params=pltpu.CompilerParams(use_tc_tiling_on_sc=False),
    scratch_types=[
        pltpu.VMEM((2, _CHK), jnp.int32),
        pltpu.VMEM((2, _CHK), jnp.int32),
        pltpu.VMEM((2, _CHK, _W), jnp.bfloat16),
        pltpu.VMEM((_RZ, _W), jnp.bfloat16),
        pltpu.VMEM_SHARED((_NPAD, _W), jnp.bfloat16),
        pltpu.SemaphoreType.DMA,
        pltpu.SemaphoreType.DMA,
    ],
)
def _sc_agg(src_h, dst_h, t0h, t1h, t2h, t3h, z_h,
            o0, o1, o2, o3,
            srcv, dstv, rows, zv, acc, gsem, ssem):
    core = lax.axis_index("c")
    s = lax.axis_index("s")
    pltpu.sync_copy(z_h, zv)
    hbpt = _NBLK // 2 // _NS
    blk0 = core * (_NBLK // 2) + s * hbpt
    rpt = _NPAD // _NS

    for th, oh in ((t0h, o0), (t1h, o1), (t2h, o2), (t3h, o3)):
        for kk in range(16):
            pltpu.sync_copy(zv, acc.at[pl.ds((s * 16 + kk) * _RZ, _RZ), :])
        plsc.subcore_barrier()

        e0 = blk0 * 128
        for b in range(2):
            pltpu.sync_copy(src_h.at[pl.ds(e0 + b * _CHK, _CHK)], srcv.at[b])
            pltpu.sync_copy(dst_h.at[pl.ds(e0 + b * _CHK, _CHK)], dstv.at[b])
            pltpu.async_copy(th.at[srcv.at[b]], rows.at[b], gsem)

        def outer(o, carry):
            for b in range(2):
                i = 2 * o + b
                pltpu.make_async_copy(th.at[srcv.at[b]], rows.at[b],
                                      gsem).wait()
                pltpu.async_copy(rows.at[b], acc.at[dstv.at[b]], ssem,
                                 add=True)
                pltpu.make_async_copy(rows.at[b], acc.at[dstv.at[b]],
                                      ssem).wait()

                @pl.when(i + 2 < _NIT)
                def _():
                    eo = e0 + (i + 2) * _CHK
                    pltpu.sync_copy(src_h.at[pl.ds(eo, _CHK)], srcv.at[b])
                    pltpu.sync_copy(dst_h.at[pl.ds(eo, _CHK)], dstv.at[b])
                    pltpu.async_copy(th.at[srcv.at[b]], rows.at[b], gsem)
            return carry

        lax.fori_loop(0, _NIT // 2, outer, 0)
        plsc.subcore_barrier()
        pltpu.sync_copy(acc.at[pl.ds(s * rpt, rpt), :],
                        oh.at[core, pl.ds(s * rpt, rpt), :])
        plsc.subcore_barrier()


_B = 1000
_NB = _N // _B


def _tc1_body(x_ref, da_ref, db_ref, w1_ref, *outs):
    os_, dv_ref = outs[:_NGRP], outs[_NGRP]
    deg = da_ref[...] + db_ref[...] + 1.0
    dinv = lax.rsqrt(deg)
    h = jnp.dot(x_ref[...], w1_ref[...], preferred_element_type=jnp.float32)
    hp = h * dinv
    dv_ref[...] = dinv
    for g in range(_NGRP):
        os_[g][...] = hp[:, g * _W:(g + 1) * _W].astype(jnp.bfloat16)


def _tc2_body(*refs):
    ags = refs[:_NGRP]
    hgs = refs[_NGRP:2 * _NGRP]
    dv_ref, b1_ref, w2_ref = refs[2 * _NGRP:2 * _NGRP + 3]
    os_ = refs[2 * _NGRP + 3:]
    dinv = dv_ref[...]
    parts = []
    for ag, hg in zip(ags, hgs):
        a = ag[...].astype(jnp.float32)
        parts.append(a[0] + a[1] + hg[...].astype(jnp.float32))
    z = jnp.concatenate(parts, axis=1) * dinv + b1_ref[...]
    z = jnp.maximum(z, 0.0)
    h2o = jnp.dot(z, w2_ref[...], preferred_element_type=jnp.float32)
    hp = h2o * dinv
    for g in range(_NGRP):
        os_[g][...] = hp[:, g * _W:(g + 1) * _W].astype(jnp.bfloat16)


def _tc3_body(*refs):
    ags = refs[:_NGRP]
    hgs = refs[_NGRP:2 * _NGRP]
    (dv_ref, b2_ref, bat_ref, lw1_ref, lb1_ref, lw2_ref, lb2_ref,
     out_ref, sums_ref, cnt_ref) = refs[2 * _NGRP:]
    i = pl.program_id(0)
    dinv = dv_ref[...]
    parts = []
    for ag, hg in zip(ags, hgs):
        a = ag[...].astype(jnp.float32)
        parts.append(a[0] + a[1] + hg[...].astype(jnp.float32))
    z = jnp.concatenate(parts, axis=1) * dinv + b2_ref[...]
    z = jnp.maximum(z, 0.0)
    gid = lax.broadcasted_iota(jnp.int32, (_B, _G), 1)
    oh = (bat_ref[...] == gid).astype(jnp.float32)
    dn = (((0,), (0,)), ((), ()))
    psum = lax.dot_general(oh, z, dn, preferred_element_type=jnp.float32)
    pcnt = lax.dot_general(oh, jnp.ones((_B, 1), jnp.float32), dn,
                           preferred_element_type=jnp.float32)

    @pl.when(i == 0)
    def _():
        sums_ref[...] = psum
        cnt_ref[...] = pcnt

    @pl.when(i > 0)
    def _():
        sums_ref[...] += psum
        cnt_ref[...] += pcnt

    @pl.when(i == _NB - 1)
    def _():
        p = sums_ref[...] / jnp.maximum(cnt_ref[...], 1.0)
        q = jnp.dot(p, lw1_ref[...], preferred_element_type=jnp.float32)
        q = jnp.maximum(q + lb1_ref[...], 0.0)
        out_ref[...] = jnp.dot(q, lw2_ref[...],
                               preferred_element_type=jnp.float32) + lb2_ref[...]


def _row_spec(w):
    return pl.BlockSpec((_B, w), lambda i: (i, 0))


def _agg_spec():
    return pl.BlockSpec((_NC, _B, _W), lambda i: (0, i, 0))


def _full_spec(shape):
    nd = len(shape)
    return pl.BlockSpec(shape, lambda i: (0,) * nd)


def kernel(x, edge_index, batch, W1, b1, W2, b2, LW1, Lb1, LW2, Lb2):
    src = edge_index[0]
    dst = edge_index[1]
    pad = _EPAD - _E
    srcf = jnp.concatenate([src, jnp.zeros((pad,), jnp.int32)])
    dstf = jnp.concatenate([dst, jnp.full((pad,), _JUNK, jnp.int32)])
    dst2 = dstf.reshape(_NBLK, 128)

    zeros_d = jnp.zeros((_RZ, _DW), jnp.float32)
    zeros_w = jnp.zeros((_RZ, _W), jnp.bfloat16)
    onehot_rows = jnp.zeros((128, _DW), jnp.float32).at[:, 0].set(1.0)

    deg_out = _sc_deg(dst2, onehot_rows, zeros_d)
    degA = deg_out[0, :_N, 0:1]
    degB = deg_out[1, :_N, 0:1]

    t1 = pl.pallas_call(
        _tc1_body,
        grid=(_NB,),
        in_specs=[_row_spec(11), _row_spec(1), _row_spec(1), _full_spec((11, _H))],
        out_specs=[_row_spec(_W)] * _NGRP + [_row_spec(1)],
        out_shape=[jax.ShapeDtypeStruct((_N, _W), jnp.bfloat16)] * _NGRP
        + [jax.ShapeDtypeStruct((_N, 1), jnp.float32)],
    )(x, degA, degB, W1)
    h1g, dinv = t1[:_NGRP], t1[_NGRP]

    ag1 = _sc_agg(srcf, dstf, *h1g, zeros_w)

    h2g = pl.pallas_call(
        _tc2_body,
        grid=(_NB,),
        in_specs=[_agg_spec()] * _NGRP + [_row_spec(_W)] * _NGRP
        + [_row_spec(1), _full_spec((1, _H)), _full_spec((_H, _H))],
        out_specs=[_row_spec(_W)] * _NGRP,
        out_shape=[jax.ShapeDtypeStruct((_N, _W), jnp.bfloat16)] * _NGRP,
    )(*ag1, *h1g, dinv, b1.reshape(1, _H), W2)

    ag2 = _sc_agg(srcf, dstf, *h2g, zeros_w)

    out = pl.pallas_call(
        _tc3_body,
        grid=(_NB,),
        in_specs=[_agg_spec()] * _NGRP + [_row_spec(_W)] * _NGRP
        + [_row_spec(1), _full_spec((1, _H)), _row_spec(1),
           _full_spec((_H, _H)), _full_spec((1, _H)),
           _full_spec((_H, 1)), _full_spec((1, 1))],
        out_specs=pl.BlockSpec((_G, 1), lambda i: (0, 0)),
        out_shape=jax.ShapeDtypeStruct((_G, 1), jnp.float32),
        scratch_shapes=[pltpu.VMEM((_G, _G), jnp.float32),
                        pltpu.VMEM((_G, 1), jnp.float32)],
    )(*ag2, *h2g, dinv, b2.reshape(1, _H), batch.reshape(_N, 1),
      LW1, Lb1.reshape(1, _H), LW2, Lb2.reshape(1, 1))

    return out


# speedup vs baseline: 12.4722x; 1.0483x over previous
import functools
import jax
import jax.numpy as jnp
from jax import lax
from jax.experimental import pallas as pl
from jax.experimental.pallas import tpu as pltpu
from jax.experimental.pallas import tpu_sc as plsc

_N = 100000
_E = 1600000
_H = 64
_G = 64
_NC, _NS = 2, 16

_EPAD = 1638400
_NBLK = _EPAD // 128
_BPT = _NBLK // (_NC * _NS)
_NPAD = 100352
_JUNK = _NPAD - 1
_RZ = 392
_W = 16
_NGRP = _H // _W
_DW = 8
_CHK = 3200
_NIT = 16

_mesh = plsc.VectorSubcoreMesh(core_axis_name="c", subcore_axis_name="s")


@functools.partial(
    pl.kernel,
    out_type=jax.ShapeDtypeStruct((_NC, _NPAD, _DW), jnp.float32),
    mesh=_mesh,
    compiler_params=pltpu.CompilerParams(use_tc_tiling_on_sc=False),
    scratch_types=[
        pltpu.VMEM((16, 128), jnp.int32),
        pltpu.VMEM((128, _DW), jnp.float32),
        pltpu.VMEM((_RZ, _DW), jnp.float32),
        pltpu.VMEM_SHARED((_NPAD, _DW), jnp.float32),
        pltpu.SemaphoreType.DMA,
    ],
)
def _sc_deg(dst_h, oh_h, z_h, out_h, idxv, ohv, zv, acc, sem):
    core = lax.axis_index("c")
    s = lax.axis_index("s")
    pltpu.sync_copy(z_h, zv)
    pltpu.sync_copy(oh_h, ohv)
    for kk in range(16):
        pltpu.sync_copy(zv, acc.at[pl.ds((s * 16 + kk) * _RZ, _RZ), :])
    plsc.subcore_barrier()
    blk0 = (core * _NS + s) * _BPT

    def body(i, carry):
        pltpu.sync_copy(dst_h.at[pl.ds(blk0 + i * 16, 16), :], idxv)
        for j in range(16):
            pltpu.sync_copy(ohv, acc.at[idxv.at[j]], add=True)
        return carry

    lax.fori_loop(0, _BPT // 16, body, 0)
    plsc.subcore_barrier()
    rpt = _NPAD // _NS
    pltpu.sync_copy(acc.at[pl.ds(s * rpt, rpt), :],
                    out_h.at[core, pl.ds(s * rpt, rpt), :])


@functools.partial(
    pl.kernel,
    out_type=[jax.ShapeDtypeStruct((_NC, _NPAD, _W), jnp.bfloat16)] * _NGRP,
    mesh=_mesh,
    compiler_params=pltpu.CompilerParams(use_tc_tiling_on_sc=False),
    scratch_types=[
        pltpu.VMEM((2, _CHK), jnp.int32),
        pltpu.VMEM((2, _CHK), jnp.int32),
        pltpu.VMEM((2, _CHK, _W), jnp.bfloat16),
        pltpu.VMEM((_RZ, _W), jnp.bfloat16),
        pltpu.VMEM_SHARED((_NPAD, _W), jnp.bfloat16),
        pltpu.SemaphoreType.DMA,
        pltpu.SemaphoreType.DMA,
    ],
)
def _sc_agg(src_h, dst_h, t0h, t1h, t2h, t3h, z_h,
            o0, o1, o2, o3,
            srcv, dstv, rows, zv, acc, gsem, ssem):
    core = lax.axis_index("c")
    s = lax.axis_index("s")
    pltpu.sync_copy(z_h, zv)
    hbpt = _NBLK // 2 // _NS
    blk0 = core * (_NBLK // 2) + s * hbpt
    rpt = _NPAD // _NS

    for th, oh in ((t0h, o0), (t1h, o1), (t2h, o2), (t3h, o3)):
        for kk in range(16):
            pltpu.sync_copy(zv, acc.at[pl.ds((s * 16 + kk) * _RZ, _RZ), :])
        plsc.subcore_barrier()

        e0 = blk0 * 128
        for b in range(2):
            pltpu.sync_copy(src_h.at[pl.ds(e0 + b * _CHK, _CHK)], srcv.at[b])
            pltpu.sync_copy(dst_h.at[pl.ds(e0 + b * _CHK, _CHK)], dstv.at[b])
            pltpu.async_copy(th.at[srcv.at[b]], rows.at[b], gsem)

        def outer(o, carry):
            for b in range(2):
                i = 2 * o + b
                pltpu.make_async_copy(th.at[srcv.at[b]], rows.at[b],
                                      gsem).wait()
                pltpu.async_copy(rows.at[b], acc.at[dstv.at[b]], ssem,
                                 add=True)
                pltpu.make_async_copy(rows.at[b], acc.at[dstv.at[b]],
                                      ssem).wait()

                @pl.when(i + 2 < _NIT)
                def _():
                    eo = e0 + (i + 2) * _CHK
                    pltpu.sync_copy(src_h.at[pl.ds(eo, _CHK)], srcv.at[b])
                    pltpu.sync_copy(dst_h.at[pl.ds(eo, _CHK)], dstv.at[b])
                    pltpu.async_copy(th.at[srcv.at[b]], rows.at[b], gsem)
            return carry

        lax.fori_loop(0, _NIT // 2, outer, 0)
        plsc.subcore_barrier()
        pltpu.sync_copy(acc.at[pl.ds(s * rpt, rpt), :],
                        oh.at[core, pl.ds(s * rpt, rpt), :])
        plsc.subcore_barrier()


_B = 2000
_NB = _N // _B


def _tc1_body(x_ref, da_ref, db_ref, w1_ref, *outs):
    os_, dv_ref = outs[:_NGRP], outs[_NGRP]
    deg = da_ref[...] + db_ref[...] + 1.0
    dinv = lax.rsqrt(deg)
    h = jnp.dot(x_ref[...], w1_ref[...], preferred_element_type=jnp.float32)
    hp = h * dinv
    dv_ref[...] = dinv
    for g in range(_NGRP):
        os_[g][...] = hp[:, g * _W:(g + 1) * _W].astype(jnp.bfloat16)


def _tc2_body(*refs):
    ags = refs[:_NGRP]
    hgs = refs[_NGRP:2 * _NGRP]
    dv_ref, b1_ref, w2_ref = refs[2 * _NGRP:2 * _NGRP + 3]
    os_ = refs[2 * _NGRP + 3:]
    dinv = dv_ref[...]
    parts = []
    for ag, hg in zip(ags, hgs):
        a = ag[...].astype(jnp.float32)
        parts.append(a[0] + a[1] + hg[...].astype(jnp.float32))
    z = jnp.concatenate(parts, axis=1) * dinv + b1_ref[...]
    z = jnp.maximum(z, 0.0)
    h2o = jnp.dot(z, w2_ref[...], preferred_element_type=jnp.float32)
    hp = h2o * dinv
    for g in range(_NGRP):
        os_[g][...] = hp[:, g * _W:(g + 1) * _W].astype(jnp.bfloat16)


def _tc3_body(*refs):
    ags = refs[:_NGRP]
    hgs = refs[_NGRP:2 * _NGRP]
    (dv_ref, b2_ref, bat_ref, lw1_ref, lb1_ref, lw2_ref, lb2_ref,
     out_ref, sums_ref, cnt_ref) = refs[2 * _NGRP:]
    i = pl.program_id(0)
    dinv = dv_ref[...]
    parts = []
    for ag, hg in zip(ags, hgs):
        a = ag[...].astype(jnp.float32)
        parts.append(a[0] + a[1] + hg[...].astype(jnp.float32))
    z = jnp.concatenate(parts, axis=1) * dinv + b2_ref[...]
    z = jnp.maximum(z, 0.0)
    gid = lax.broadcasted_iota(jnp.int32, (_B, _G), 1)
    oh = (bat_ref[...] == gid).astype(jnp.float32)
    dn = (((0,), (0,)), ((), ()))
    psum = lax.dot_general(oh, z, dn, preferred_element_type=jnp.float32)
    pcnt = lax.dot_general(oh, jnp.ones((_B, 1), jnp.float32), dn,
                           preferred_element_type=jnp.float32)

    @pl.when(i == 0)
    def _():
        sums_ref[...] = psum
        cnt_ref[...] = pcnt

    @pl.when(i > 0)
    def _():
        sums_ref[...] += psum
        cnt_ref[...] += pcnt

    @pl.when(i == _NB - 1)
    def _():
        p = sums_ref[...] / jnp.maximum(cnt_ref[...], 1.0)
        q = jnp.dot(p, lw1_ref[...], preferred_element_type=jnp.float32)
        q = jnp.maximum(q + lb1_ref[...], 0.0)
        out_ref[...] = jnp.dot(q, lw2_ref[...],
                               preferred_element_type=jnp.float32) + lb2_ref[...]


def _row_spec(w):
    return pl.BlockSpec((_B, w), lambda i: (i, 0))


def _agg_spec():
    return pl.BlockSpec((_NC, _B, _W), lambda i: (0, i, 0))


def _full_spec(shape):
    nd = len(shape)
    return pl.BlockSpec(shape, lambda i: (0,) * nd)


def kernel(x, edge_index, batch, W1, b1, W2, b2, LW1, Lb1, LW2, Lb2):
    src = edge_index[0]
    dst = edge_index[1]
    pad = _EPAD - _E
    srcf = jnp.concatenate([src, jnp.zeros((pad,), jnp.int32)])
    dstf = jnp.concatenate([dst, jnp.full((pad,), _JUNK, jnp.int32)])
    dst2 = dstf.reshape(_NBLK, 128)

    zeros_d = jnp.zeros((_RZ, _DW), jnp.float32)
    zeros_w = jnp.zeros((_RZ, _W), jnp.bfloat16)
    onehot_rows = jnp.zeros((128, _DW), jnp.float32).at[:, 0].set(1.0)

    deg_out = _sc_deg(dst2, onehot_rows, zeros_d)
    degA = deg_out[0, :_N, 0:1]
    degB = deg_out[1, :_N, 0:1]

    t1 = pl.pallas_call(
        _tc1_body,
        grid=(_NB,),
        in_specs=[_row_spec(11), _row_spec(1), _row_spec(1), _full_spec((11, _H))],
        out_specs=[_row_spec(_W)] * _NGRP + [_row_spec(1)],
        out_shape=[jax.ShapeDtypeStruct((_N, _W), jnp.bfloat16)] * _NGRP
        + [jax.ShapeDtypeStruct((_N, 1), jnp.float32)],
    )(x, degA, degB, W1)
    h1g, dinv = t1[:_NGRP], t1[_NGRP]

    ag1 = _sc_agg(srcf, dstf, *h1g, zeros_w)

    h2g = pl.pallas_call(
        _tc2_body,
        grid=(_NB,),
        in_specs=[_agg_spec()] * _NGRP + [_row_spec(_W)] * _NGRP
        + [_row_spec(1), _full_spec((1, _H)), _full_spec((_H, _H))],
        out_specs=[_row_spec(_W)] * _NGRP,
        out_shape=[jax.ShapeDtypeStruct((_N, _W), jnp.bfloat16)] * _NGRP,
    )(*ag1, *h1g, dinv, b1.reshape(1, _H), W2)

    ag2 = _sc_agg(srcf, dstf, *h2g, zeros_w)

    out = pl.pallas_call(
        _tc3_body,
        grid=(_NB,),
        in_specs=[_agg_spec()] * _NGRP + [_row_spec(_W)] * _NGRP
        + [_row_spec(1), _full_spec((1, _H)), _row_spec(1),
           _full_spec((_H, _H)), _full_spec((1, _H)),
           _full_spec((_H, 1)), _full_spec((1, 1))],
        out_specs=pl.BlockSpec((_G, 1), lambda i: (0, 0)),
        out_shape=jax.ShapeDtypeStruct((_G, 1), jnp.float32),
        scratch_shapes=[pltpu.VMEM((_G, _G), jnp.float32),
                        pltpu.VMEM((_G, 1), jnp.float32)],
    )(*ag2, *h2g, dinv, b2.reshape(1, _H), batch.reshape(_N, 1),
      LW1, Lb1.reshape(1, _H), LW2, Lb2.reshape(1, 1))

    return out


# speedup vs baseline: 12.5476x; 1.0060x over previous
import functools
import jax
import jax.numpy as jnp
from jax import lax
from jax.experimental import pallas as pl
from jax.experimental.pallas import tpu as pltpu
from jax.experimental.pallas import tpu_sc as plsc

_N = 100000
_E = 1600000
_H = 64
_G = 64
_NC, _NS = 2, 16

_EPAD = 1638400
_NBLK = _EPAD // 128
_BPT = _NBLK // (_NC * _NS)
_NPAD = 100352
_JUNK = _NPAD - 1
_RZ = 392
_W = 16
_NGRP = _H // _W
_DW = 8
_CHK = 3200
_NIT = 16

_mesh = plsc.VectorSubcoreMesh(core_axis_name="c", subcore_axis_name="s")


@functools.partial(
    pl.kernel,
    out_type=jax.ShapeDtypeStruct((_NC, _NPAD, _DW), jnp.float32),
    mesh=_mesh,
    compiler_params=pltpu.CompilerParams(use_tc_tiling_on_sc=False),
    scratch_types=[
        pltpu.VMEM((16, 128), jnp.int32),
        pltpu.VMEM((128, _DW), jnp.float32),
        pltpu.VMEM((_RZ, _DW), jnp.float32),
        pltpu.VMEM_SHARED((_NPAD, _DW), jnp.float32),
        pltpu.SemaphoreType.DMA,
    ],
)
def _sc_deg(dst_h, oh_h, z_h, out_h, idxv, ohv, zv, acc, sem):
    core = lax.axis_index("c")
    s = lax.axis_index("s")
    pltpu.sync_copy(z_h, zv)
    pltpu.sync_copy(oh_h, ohv)
    for kk in range(16):
        pltpu.sync_copy(zv, acc.at[pl.ds((s * 16 + kk) * _RZ, _RZ), :])
    plsc.subcore_barrier()
    blk0 = (core * _NS + s) * _BPT

    def body(i, carry):
        pltpu.sync_copy(dst_h.at[pl.ds(blk0 + i * 16, 16), :], idxv)
        for j in range(16):
            pltpu.sync_copy(ohv, acc.at[idxv.at[j]], add=True)
        return carry

    lax.fori_loop(0, _BPT // 16, body, 0)
    plsc.subcore_barrier()
    rpt = _NPAD // _NS
    pltpu.sync_copy(acc.at[pl.ds(s * rpt, rpt), :],
                    out_h.at[core, pl.ds(s * rpt, rpt), :])


@functools.partial(
    pl.kernel,
    out_type=[jax.ShapeDtypeStruct((_NC, _NPAD, _W), jnp.bfloat16)] * _NGRP,
    mesh=_mesh,
    compiler_params=pltpu.CompilerParams(use_tc_tiling_on_sc=False),
    scratch_types=[
        pltpu.VMEM((2, _CHK), jnp.int32),
        pltpu.VMEM((2, _CHK), jnp.int32),
        pltpu.VMEM((2, _CHK, _W), jnp.bfloat16),
        pltpu.VMEM((_RZ * 4, _W), jnp.bfloat16),
        pltpu.VMEM_SHARED((_NPAD, _W), jnp.bfloat16),
        pltpu.SemaphoreType.DMA,
        pltpu.SemaphoreType.DMA,
    ],
)
def _sc_agg(src_h, dst_h, t0h, t1h, t2h, t3h, z_h,
            o0, o1, o2, o3,
            srcv, dstv, rows, zv, acc, gsem, ssem):
    core = lax.axis_index("c")
    s = lax.axis_index("s")
    pltpu.sync_copy(z_h, zv)
    hbpt = _NBLK // 2 // _NS
    blk0 = core * (_NBLK // 2) + s * hbpt
    rpt = _NPAD // _NS

    for th, oh in ((t0h, o0), (t1h, o1), (t2h, o2), (t3h, o3)):
        for kk in range(4):
            pltpu.sync_copy(
                zv, acc.at[pl.ds((s * 4 + kk) * (_RZ * 4), _RZ * 4), :])
        plsc.subcore_barrier()

        e0 = blk0 * 128
        for b in range(2):
            pltpu.sync_copy(src_h.at[pl.ds(e0 + b * _CHK, _CHK)], srcv.at[b])
            pltpu.sync_copy(dst_h.at[pl.ds(e0 + b * _CHK, _CHK)], dstv.at[b])
            pltpu.async_copy(th.at[srcv.at[b]], rows.at[b], gsem)

        def outer(o, carry):
            for b in range(2):
                i = 2 * o + b
                pltpu.make_async_copy(th.at[srcv.at[b]], rows.at[b],
                                      gsem).wait()
                pltpu.async_copy(rows.at[b], acc.at[dstv.at[b]], ssem,
                                 add=True)
                pltpu.make_async_copy(rows.at[b], acc.at[dstv.at[b]],
                                      ssem).wait()

                @pl.when(i + 2 < _NIT)
                def _():
                    eo = e0 + (i + 2) * _CHK
                    pltpu.sync_copy(src_h.at[pl.ds(eo, _CHK)], srcv.at[b])
                    pltpu.sync_copy(dst_h.at[pl.ds(eo, _CHK)], dstv.at[b])
                    pltpu.async_copy(th.at[srcv.at[b]], rows.at[b], gsem)
            return carry

        lax.fori_loop(0, _NIT // 2, outer, 0)
        plsc.subcore_barrier()
        pltpu.sync_copy(acc.at[pl.ds(s * rpt, rpt), :],
                        oh.at[core, pl.ds(s * rpt, rpt), :])
        plsc.subcore_barrier()


_B = 2000
_NB = _N // _B


def _tc1_body(x_ref, da_ref, db_ref, w1_ref, *outs):
    os_, dv_ref = outs[:_NGRP], outs[_NGRP]
    deg = da_ref[...] + db_ref[...] + 1.0
    dinv = lax.rsqrt(deg)
    h = jnp.dot(x_ref[...], w1_ref[...], preferred_element_type=jnp.float32)
    hp = h * dinv
    dv_ref[...] = dinv
    for g in range(_NGRP):
        os_[g][...] = hp[:, g * _W:(g + 1) * _W].astype(jnp.bfloat16)


def _tc2_body(*refs):
    ags = refs[:_NGRP]
    hgs = refs[_NGRP:2 * _NGRP]
    dv_ref, b1_ref, w2_ref = refs[2 * _NGRP:2 * _NGRP + 3]
    os_ = refs[2 * _NGRP + 3:]
    dinv = dv_ref[...]
    parts = []
    for ag, hg in zip(ags, hgs):
        a = ag[...].astype(jnp.float32)
        parts.append(a[0] + a[1] + hg[...].astype(jnp.float32))
    z = jnp.concatenate(parts, axis=1) * dinv + b1_ref[...]
    z = jnp.maximum(z, 0.0)
    h2o = jnp.dot(z, w2_ref[...], preferred_element_type=jnp.float32)
    hp = h2o * dinv
    for g in range(_NGRP):
        os_[g][...] = hp[:, g * _W:(g + 1) * _W].astype(jnp.bfloat16)


def _tc3_body(*refs):
    ags = refs[:_NGRP]
    hgs = refs[_NGRP:2 * _NGRP]
    (dv_ref, b2_ref, bat_ref, lw1_ref, lb1_ref, lw2_ref, lb2_ref,
     out_ref, sums_ref, cnt_ref) = refs[2 * _NGRP:]
    i = pl.program_id(0)
    dinv = dv_ref[...]
    parts = []
    for ag, hg in zip(ags, hgs):
        a = ag[...].astype(jnp.float32)
        parts.append(a[0] + a[1] + hg[...].astype(jnp.float32))
    z = jnp.concatenate(parts, axis=1) * dinv + b2_ref[...]
    z = jnp.maximum(z, 0.0)
    gid = lax.broadcasted_iota(jnp.int32, (_B, _G), 1)
    oh = (bat_ref[...] == gid).astype(jnp.float32)
    dn = (((0,), (0,)), ((), ()))
    psum = lax.dot_general(oh, z, dn, preferred_element_type=jnp.float32)
    pcnt = lax.dot_general(oh, jnp.ones((_B, 1), jnp.float32), dn,
                           preferred_element_type=jnp.float32)

    @pl.when(i == 0)
    def _():
        sums_ref[...] = psum
        cnt_ref[...] = pcnt

    @pl.when(i > 0)
    def _():
        sums_ref[...] += psum
        cnt_ref[...] += pcnt

    @pl.when(i == _NB - 1)
    def _():
        p = sums_ref[...] / jnp.maximum(cnt_ref[...], 1.0)
        q = jnp.dot(p, lw1_ref[...], preferred_element_type=jnp.float32)
        q = jnp.maximum(q + lb1_ref[...], 0.0)
        out_ref[...] = jnp.dot(q, lw2_ref[...],
                               preferred_element_type=jnp.float32) + lb2_ref[...]


def _row_spec(w):
    return pl.BlockSpec((_B, w), lambda i: (i, 0))


def _agg_spec():
    return pl.BlockSpec((_NC, _B, _W), lambda i: (0, i, 0))


def _full_spec(shape):
    nd = len(shape)
    return pl.BlockSpec(shape, lambda i: (0,) * nd)


def kernel(x, edge_index, batch, W1, b1, W2, b2, LW1, Lb1, LW2, Lb2):
    src = edge_index[0]
    dst = edge_index[1]
    pad = _EPAD - _E
    srcf = jnp.concatenate([src, jnp.zeros((pad,), jnp.int32)])
    dstf = jnp.concatenate([dst, jnp.full((pad,), _JUNK, jnp.int32)])
    dst2 = dstf.reshape(_NBLK, 128)

    zeros_d = jnp.zeros((_RZ, _DW), jnp.float32)
    zeros_w = jnp.zeros((_RZ * 4, _W), jnp.bfloat16)
    onehot_rows = jnp.zeros((128, _DW), jnp.float32).at[:, 0].set(1.0)

    deg_out = _sc_deg(dst2, onehot_rows, zeros_d)
    degA = deg_out[0, :_N, 0:1]
    degB = deg_out[1, :_N, 0:1]

    t1 = pl.pallas_call(
        _tc1_body,
        grid=(_NB,),
        in_specs=[_row_spec(11), _row_spec(1), _row_spec(1), _full_spec((11, _H))],
        out_specs=[_row_spec(_W)] * _NGRP + [_row_spec(1)],
        out_shape=[jax.ShapeDtypeStruct((_N, _W), jnp.bfloat16)] * _NGRP
        + [jax.ShapeDtypeStruct((_N, 1), jnp.float32)],
    )(x, degA, degB, W1)
    h1g, dinv = t1[:_NGRP], t1[_NGRP]

    ag1 = _sc_agg(srcf, dstf, *h1g, zeros_w)

    h2g = pl.pallas_call(
        _tc2_body,
        grid=(_NB,),
        in_specs=[_agg_spec()] * _NGRP + [_row_spec(_W)] * _NGRP
        + [_row_spec(1), _full_spec((1, _H)), _full_spec((_H, _H))],
        out_specs=[_row_spec(_W)] * _NGRP,
        out_shape=[jax.ShapeDtypeStruct((_N, _W), jnp.bfloat16)] * _NGRP,
    )(*ag1, *h1g, dinv, b1.reshape(1, _H), W2)

    ag2 = _sc_agg(srcf, dstf, *h2g, zeros_w)

    out = pl.pallas_call(
        _tc3_body,
        grid=(_NB,),
        in_specs=[_agg_spec()] * _NGRP + [_row_spec(_W)] * _NGRP
        + [_row_spec(1), _full_spec((1, _H)), _row_spec(1),
           _full_spec((_H, _H)), _full_spec((1, _H)),
           _full_spec((_H, 1)), _full_spec((1, 1))],
        out_specs=pl.BlockSpec((_G, 1), lambda i: (0, 0)),
        out_shape=jax.ShapeDtypeStruct((_G, 1), jnp.float32),
        scratch_shapes=[pltpu.VMEM((_G, _G), jnp.float32),
                        pltpu.VMEM((_G, 1), jnp.float32)],
    )(*ag2, *h2g, dinv, b2.reshape(1, _H), batch.reshape(_N, 1),
      LW1, Lb1.reshape(1, _H), LW2, Lb2.reshape(1, 1))

    return out
